# baseline (speedup 1.0000x reference)
import jax
import jax.numpy as jnp
from jax import lax
from jax.experimental import pallas as pl
from jax.experimental.pallas import tpu as pltpu
from jax.experimental.pallas import tpu_sc as plsc

N_NODES = 10000
N_PAD = 10240
E = 320000
D = 128
N_CLASS = 40

NC = 2
NS = 16
EPT = E // (NC * NS)
CHUNK = 80
NCHUNK = EPT // CHUNK
SL = N_PAD // NS

ROW_BLK = 2560
TC_GRID = N_PAD // ROW_BLK


DEG_W = 128


def _deg_body(dst2_hbm, zeros_hbm, ones_hbm, deg_out,
              dstbuf, onesbuf, degacc, sem):
    c = lax.axis_index("c")
    s = lax.axis_index("s")
    g = c * NS + s

    pltpu.sync_copy(ones_hbm, onesbuf)
    pltpu.sync_copy(dst2_hbm.at[g], dstbuf)
    pltpu.sync_copy(zeros_hbm.at[pl.ds(s * SL, SL)],
                    degacc.at[pl.ds(s * SL, SL)])
    plsc.subcore_barrier()

    def fire_body(j, _):
        pltpu.async_copy(onesbuf, degacc.at[dstbuf.at[j]], sem, add=True)
        return 0
    lax.fori_loop(0, NCHUNK, fire_body, 0)

    def drain_body(j, _):
        pltpu.make_async_copy(onesbuf, degacc.at[dstbuf.at[j]], sem).wait()
        return 0
    lax.fori_loop(0, NCHUNK, drain_body, 0)

    plsc.subcore_barrier()
    pltpu.sync_copy(degacc.at[pl.ds(s * SL, SL)],
                    deg_out.at[pl.ds(c * N_PAD + s * SL, SL)])


def _deg_kernel(dst2, zeros_pad, ones_rows):
    mesh = plsc.VectorSubcoreMesh(core_axis_name="c", subcore_axis_name="s")
    return pl.kernel(
        _deg_body,
        out_type=jax.ShapeDtypeStruct((NC * N_PAD, DEG_W), jnp.float32),
        mesh=mesh,
        scratch_types=[
            pltpu.VMEM((NCHUNK, CHUNK), jnp.int32),
            pltpu.VMEM((CHUNK, DEG_W), jnp.float32),
            pltpu.VMEM_SHARED((N_PAD, DEG_W), jnp.float32),
            pltpu.SemaphoreType.DMA,
        ],
    )(dst2, zeros_pad, ones_rows)


def _agg_body(src1_hbm, dst2_hbm, mh_hbm, zeros_hbm, out_hbm,
              srcbuf, dstbuf, rows_a, rows_b, acc,
              sem_ga, sem_gb, sem_sa, sem_sb):
    c = lax.axis_index("c")
    s = lax.axis_index("s")
    g = c * NS + s

    pltpu.sync_copy(src1_hbm.at[pl.ds(g * EPT, EPT)], srcbuf)
    pltpu.sync_copy(dst2_hbm.at[g], dstbuf)
    pltpu.sync_copy(zeros_hbm.at[pl.ds(s * SL, SL)], acc.at[pl.ds(s * SL, SL)])
    plsc.subcore_barrier()

    HALF = CHUNK // 2

    def fire_g(k, rbuf, sem):
        pltpu.async_copy(mh_hbm.at[srcbuf.at[pl.ds(k * CHUNK, HALF)]],
                         rbuf.at[pl.ds(0, HALF)], sem)
        pltpu.async_copy(mh_hbm.at[srcbuf.at[pl.ds(k * CHUNK + HALF, HALF)]],
                         rbuf.at[pl.ds(HALF, HALF)], sem)

    def wait_g(rbuf, sem):
        pltpu.make_async_copy(mh_hbm.at[srcbuf.at[pl.ds(0, HALF)]],
                              rbuf.at[pl.ds(0, HALF)], sem).wait()
        pltpu.make_async_copy(mh_hbm.at[srcbuf.at[pl.ds(0, HALF)]],
                              rbuf.at[pl.ds(HALF, HALF)], sem).wait()

    def fire_s(k, rbuf, sem):
        pltpu.async_copy(rbuf, acc.at[dstbuf.at[k]], sem, add=True)

    def wait_s(rbuf, sem):
        pltpu.make_async_copy(rbuf, acc.at[dstbuf.at[0]], sem).wait()

    fire_g(0, rows_a, sem_ga)
    fire_g(1, rows_b, sem_gb)

    def pair_body(i, _):
        k = 2 * i
        wait_g(rows_a, sem_ga)
        fire_s(k, rows_a, sem_sa)

        @pl.when(k + 1 < NCHUNK)
        def _():
            wait_g(rows_b, sem_gb)
            fire_s(k + 1, rows_b, sem_sb)

        @pl.when(k + 2 < NCHUNK)
        def _():
            wait_s(rows_a, sem_sa)
            fire_g(k + 2, rows_a, sem_ga)

        @pl.when(k + 3 < NCHUNK)
        def _():
            wait_s(rows_b, sem_sb)
            fire_g(k + 3, rows_b, sem_gb)
        return 0
    lax.fori_loop(0, (NCHUNK + 1) // 2, pair_body, 0)

    wait_s(rows_a, sem_sa)
    wait_s(rows_b, sem_sb)

    plsc.subcore_barrier()
    pltpu.sync_copy(acc.at[pl.ds(s * SL, SL)],
                    out_hbm.at[pl.ds(c * N_PAD + s * SL, SL)])


def _agg_kernel(src1, dst2, mh, zeros_pad):
    mesh = plsc.VectorSubcoreMesh(core_axis_name="c", subcore_axis_name="s")
    return pl.kernel(
        _agg_body,
        out_type=jax.ShapeDtypeStruct((NC * N_PAD, D), jnp.float32),
        mesh=mesh,
        scratch_types=[
            pltpu.VMEM((EPT,), jnp.int32),
            pltpu.VMEM((NCHUNK, CHUNK), jnp.int32),
            pltpu.VMEM((CHUNK, D), jnp.float32),
            pltpu.VMEM((CHUNK, D), jnp.float32),
            pltpu.VMEM_SHARED((N_PAD, D), jnp.float32),
            pltpu.SemaphoreType.DMA,
            pltpu.SemaphoreType.DMA,
            pltpu.SemaphoreType.DMA,
            pltpu.SemaphoreType.DMA,
        ],
    )(src1, dst2, mh, zeros_pad)


def _m0_body(x_ref, w_ref, m_ref):
    m_ref[...] = jnp.dot(x_ref[...], w_ref[...],
                         preferred_element_type=jnp.float32)


def _tc_m0(x_pad, W0):
    return pl.pallas_call(
        _m0_body,
        grid=(TC_GRID,),
        in_specs=[
            pl.BlockSpec((ROW_BLK, D), lambda i: (i, 0)),
            pl.BlockSpec((D, D), lambda i: (0, 0)),
        ],
        out_specs=[pl.BlockSpec((ROW_BLK, D), lambda i: (i, 0))],
        out_shape=[jax.ShapeDtypeStruct((N_PAD, D), jnp.float32)],
    )(x_pad, W0)


def _scale_body(m_ref, degp_ref, mh_ref, dinv_ref):
    deg = degp_ref[0, :, 0] + degp_ref[1, :, 0] + 1.0
    dinv = lax.rsqrt(deg)[:, None]
    mh_ref[...] = m_ref[...] * dinv
    dinv_ref[...] = dinv


def _tc_scale(m0, degp):
    return pl.pallas_call(
        _scale_body,
        grid=(TC_GRID,),
        in_specs=[
            pl.BlockSpec((ROW_BLK, D), lambda i: (i, 0)),
            pl.BlockSpec((NC, ROW_BLK, DEG_W), lambda i: (0, i, 0)),
        ],
        out_specs=[
            pl.BlockSpec((ROW_BLK, D), lambda i: (i, 0)),
            pl.BlockSpec((ROW_BLK, 1), lambda i: (i, 0)),
        ],
        out_shape=[
            jax.ShapeDtypeStruct((N_PAD, D), jnp.float32),
            jax.ShapeDtypeStruct((N_PAD, 1), jnp.float32),
        ],
    )(m0, degp)


def _mid_body(p_ref, m_prev_ref, dinv_ref, b_ref, jk_ref, w_ref,
              m_ref, mh_ref, jk_out_ref):
    dinv = dinv_ref[...]
    agg = p_ref[0] + p_ref[1]
    xl = jnp.maximum(dinv * agg + dinv * dinv * m_prev_ref[...]
                     + b_ref[...][None, :], 0.0)
    jk_out_ref[...] = jnp.maximum(jk_ref[...], xl)
    m = jnp.dot(xl, w_ref[...], preferred_element_type=jnp.float32)
    m_ref[...] = m
    mh_ref[...] = m * dinv


def _tc_mid(p, m_prev, dinv, b, jk, W):
    return pl.pallas_call(
        _mid_body,
        grid=(TC_GRID,),
        in_specs=[
            pl.BlockSpec((NC, ROW_BLK, D), lambda i: (0, i, 0)),
            pl.BlockSpec((ROW_BLK, D), lambda i: (i, 0)),
            pl.BlockSpec((ROW_BLK, 1), lambda i: (i, 0)),
            pl.BlockSpec((D,), lambda i: (0,)),
            pl.BlockSpec((ROW_BLK, D), lambda i: (i, 0)),
            pl.BlockSpec((D, D), lambda i: (0, 0)),
        ],
        out_specs=[
            pl.BlockSpec((ROW_BLK, D), lambda i: (i, 0)),
            pl.BlockSpec((ROW_BLK, D), lambda i: (i, 0)),
            pl.BlockSpec((ROW_BLK, D), lambda i: (i, 0)),
        ],
        out_shape=[
            jax.ShapeDtypeStruct((N_PAD, D), jnp.float32),
            jax.ShapeDtypeStruct((N_PAD, D), jnp.float32),
            jax.ShapeDtypeStruct((N_PAD, D), jnp.float32),
        ],
    )(p, m_prev, dinv, b, jk, W)


def _post_body(p_ref, m_prev_ref, dinv_ref, b_ref, jk_ref, wo_ref, bo_ref,
               out_ref):
    dinv = dinv_ref[...]
    agg = p_ref[0] + p_ref[1]
    xl = jnp.maximum(dinv * agg + dinv * dinv * m_prev_ref[...]
                     + b_ref[...][None, :], 0.0)
    jk = jnp.maximum(jk_ref[...], xl)
    out_ref[...] = (jnp.dot(jk, wo_ref[...], preferred_element_type=jnp.float32)
                    + bo_ref[...][None, :])


def _tc_post(p, m_prev, dinv, b, jk, Wo, bo):
    return pl.pallas_call(
        _post_body,
        grid=(TC_GRID,),
        in_specs=[
            pl.BlockSpec((NC, ROW_BLK, D), lambda i: (0, i, 0)),
            pl.BlockSpec((ROW_BLK, D), lambda i: (i, 0)),
            pl.BlockSpec((ROW_BLK, 1), lambda i: (i, 0)),
            pl.BlockSpec((D,), lambda i: (0,)),
            pl.BlockSpec((ROW_BLK, D), lambda i: (i, 0)),
            pl.BlockSpec((D, N_CLASS), lambda i: (0, 0)),
            pl.BlockSpec((N_CLASS,), lambda i: (0,)),
        ],
        out_specs=[pl.BlockSpec((ROW_BLK, N_CLASS), lambda i: (i, 0))],
        out_shape=[jax.ShapeDtypeStruct((N_PAD, N_CLASS), jnp.float32)],
    )(p, m_prev, dinv, b, jk, Wo, bo)


def kernel(x, edge_index, W0, b0, W1, b1, W2, b2, Wo, bo):
    src1 = edge_index[0]
    dst2 = edge_index[1].reshape(NC * NS, NCHUNK, CHUNK)
    x_pad = jnp.pad(x, ((0, N_PAD - N_NODES), (0, 0)))
    zeros_pad = jnp.zeros((N_PAD, D), jnp.float32)
    ones_rows = jnp.ones((CHUNK, DEG_W), jnp.float32)
    jk0 = jnp.zeros((N_PAD, D), jnp.float32)

    m0 = _tc_m0(x_pad, W0)[0]
    degp = _deg_kernel(dst2, zeros_pad, ones_rows).reshape(NC, N_PAD, DEG_W)
    mh0, dinv = _tc_scale(m0, degp)

    p1 = _agg_kernel(src1, dst2, mh0, zeros_pad).reshape(NC, N_PAD, D)
    m1, mh1, jk1 = _tc_mid(p1, m0, dinv, b0, jk0, W1)

    p2 = _agg_kernel(src1, dst2, mh1, zeros_pad).reshape(NC, N_PAD, D)
    m2, mh2, jk2 = _tc_mid(p2, m1, dinv, b1, jk1, W2)

    p3 = _agg_kernel(src1, dst2, mh2, zeros_pad).reshape(NC, N_PAD, D)
    out = _tc_post(p3, m2, dinv, b2, jk2, Wo, bo)[0]

    return out[:N_NODES]


# speedup vs baseline: 1.2019x; 1.2019x over previous
import jax
import jax.numpy as jnp
from jax import lax
from jax.experimental import pallas as pl
from jax.experimental.pallas import tpu as pltpu
from jax.experimental.pallas import tpu_sc as plsc

N_NODES = 10000
N_PAD = 10240
E = 320000
D = 128
N_CLASS = 40

NC = 2
NS = 16
EPT = E // (NC * NS)
CHUNK = 80
NCHUNK = EPT // CHUNK
SL = N_PAD // NS

ROW_BLK = 2560
TC_GRID = N_PAD // ROW_BLK


DEG_W = 128


def _deg_body(dst2_hbm, zeros_hbm, ones_hbm, deg_out,
              dstbuf, onesbuf, degacc, sem):
    c = lax.axis_index("c")
    s = lax.axis_index("s")
    g = c * NS + s

    pltpu.sync_copy(ones_hbm, onesbuf)
    pltpu.sync_copy(dst2_hbm.at[g], dstbuf)
    pltpu.sync_copy(zeros_hbm.at[pl.ds(s * SL, SL)],
                    degacc.at[pl.ds(s * SL, SL)])
    plsc.subcore_barrier()

    def fire_body(j, _):
        pltpu.async_copy(onesbuf, degacc.at[dstbuf.at[j]], sem, add=True)
        return 0
    lax.fori_loop(0, NCHUNK, fire_body, 0)

    def drain_body(j, _):
        pltpu.make_async_copy(onesbuf, degacc.at[dstbuf.at[j]], sem).wait()
        return 0
    lax.fori_loop(0, NCHUNK, drain_body, 0)

    plsc.subcore_barrier()
    pltpu.sync_copy(degacc.at[pl.ds(s * SL, SL)],
                    deg_out.at[pl.ds(c * N_PAD + s * SL, SL)])


def _deg_kernel(dst2, zeros_pad, ones_rows):
    mesh = plsc.VectorSubcoreMesh(core_axis_name="c", subcore_axis_name="s")
    return pl.kernel(
        _deg_body,
        out_type=jax.ShapeDtypeStruct((NC * N_PAD, DEG_W), jnp.float32),
        mesh=mesh,
        scratch_types=[
            pltpu.VMEM((NCHUNK, CHUNK), jnp.int32),
            pltpu.VMEM((CHUNK, DEG_W), jnp.float32),
            pltpu.VMEM_SHARED((N_PAD, DEG_W), jnp.float32),
            pltpu.SemaphoreType.DMA,
        ],
    )(dst2, zeros_pad, ones_rows)


def _agg_body(src1_hbm, dst2_hbm, mh_hbm, zeros_hbm, out_hbm,
              srcbuf, dstbuf, rows_a, rows_b, acc,
              sem_ga, sem_gb, sem_sa, sem_sb):
    c = lax.axis_index("c")
    s = lax.axis_index("s")
    g = c * NS + s

    pltpu.sync_copy(src1_hbm.at[pl.ds(g * EPT, EPT)], srcbuf)
    pltpu.sync_copy(dst2_hbm.at[g], dstbuf)
    pltpu.sync_copy(zeros_hbm.at[pl.ds(s * SL, SL)], acc.at[pl.ds(s * SL, SL)])
    plsc.subcore_barrier()

    HALF = CHUNK // 2

    def fire_g(k, rbuf, sem):
        pltpu.async_copy(mh_hbm.at[srcbuf.at[pl.ds(k * CHUNK, HALF)]],
                         rbuf.at[pl.ds(0, HALF)], sem)
        pltpu.async_copy(mh_hbm.at[srcbuf.at[pl.ds(k * CHUNK + HALF, HALF)]],
                         rbuf.at[pl.ds(HALF, HALF)], sem)

    def wait_g(rbuf, sem):
        pltpu.make_async_copy(mh_hbm.at[srcbuf.at[pl.ds(0, HALF)]],
                              rbuf.at[pl.ds(0, HALF)], sem).wait()
        pltpu.make_async_copy(mh_hbm.at[srcbuf.at[pl.ds(0, HALF)]],
                              rbuf.at[pl.ds(HALF, HALF)], sem).wait()

    def scatter(k, rbuf):
        pltpu.sync_copy(rbuf, acc.at[dstbuf.at[k]], add=True)

    fire_g(0, rows_a, sem_ga)

    def pair_body(i, _):
        k = 2 * i
        fire_g(k + 1, rows_b, sem_gb)
        wait_g(rows_a, sem_ga)
        scatter(k, rows_a)
        fire_g(k + 2, rows_a, sem_ga)
        wait_g(rows_b, sem_gb)
        scatter(k + 1, rows_b)
        return 0
    lax.fori_loop(0, (NCHUNK - 1) // 2, pair_body, 0)

    wait_g(rows_a, sem_ga)
    scatter(NCHUNK - 1, rows_a)

    plsc.subcore_barrier()
    pltpu.sync_copy(acc.at[pl.ds(s * SL, SL)],
                    out_hbm.at[pl.ds(c * N_PAD + s * SL, SL)])


def _agg_kernel(src1, dst2, mh, zeros_pad):
    mesh = plsc.VectorSubcoreMesh(core_axis_name="c", subcore_axis_name="s")
    return pl.kernel(
        _agg_body,
        out_type=jax.ShapeDtypeStruct((NC * N_PAD, D), jnp.float32),
        mesh=mesh,
        scratch_types=[
            pltpu.VMEM((EPT,), jnp.int32),
            pltpu.VMEM((NCHUNK, CHUNK), jnp.int32),
            pltpu.VMEM((CHUNK, D), jnp.float32),
            pltpu.VMEM((CHUNK, D), jnp.float32),
            pltpu.VMEM_SHARED((N_PAD, D), jnp.float32),
            pltpu.SemaphoreType.DMA,
            pltpu.SemaphoreType.DMA,
            pltpu.SemaphoreType.DMA,
            pltpu.SemaphoreType.DMA,
        ],
    )(src1, dst2, mh, zeros_pad)


def _m0_body(x_ref, w_ref, m_ref):
    m_ref[...] = jnp.dot(x_ref[...], w_ref[...],
                         preferred_element_type=jnp.float32)


def _tc_m0(x_pad, W0):
    return pl.pallas_call(
        _m0_body,
        grid=(TC_GRID,),
        in_specs=[
            pl.BlockSpec((ROW_BLK, D), lambda i: (i, 0)),
            pl.BlockSpec((D, D), lambda i: (0, 0)),
        ],
        out_specs=[pl.BlockSpec((ROW_BLK, D), lambda i: (i, 0))],
        out_shape=[jax.ShapeDtypeStruct((N_PAD, D), jnp.float32)],
    )(x_pad, W0)


def _scale_body(m_ref, degp_ref, mh_ref, dinv_ref):
    deg = degp_ref[0, :, 0] + degp_ref[1, :, 0] + 1.0
    dinv = lax.rsqrt(deg)[:, None]
    mh_ref[...] = m_ref[...] * dinv
    dinv_ref[...] = dinv


def _tc_scale(m0, degp):
    return pl.pallas_call(
        _scale_body,
        grid=(TC_GRID,),
        in_specs=[
            pl.BlockSpec((ROW_BLK, D), lambda i: (i, 0)),
            pl.BlockSpec((NC, ROW_BLK, DEG_W), lambda i: (0, i, 0)),
        ],
        out_specs=[
            pl.BlockSpec((ROW_BLK, D), lambda i: (i, 0)),
            pl.BlockSpec((ROW_BLK, 1), lambda i: (i, 0)),
        ],
        out_shape=[
            jax.ShapeDtypeStruct((N_PAD, D), jnp.float32),
            jax.ShapeDtypeStruct((N_PAD, 1), jnp.float32),
        ],
    )(m0, degp)


def _mid_body(p_ref, m_prev_ref, dinv_ref, b_ref, jk_ref, w_ref,
              m_ref, mh_ref, jk_out_ref):
    dinv = dinv_ref[...]
    agg = p_ref[0] + p_ref[1]
    xl = jnp.maximum(dinv * agg + dinv * dinv * m_prev_ref[...]
                     + b_ref[...][None, :], 0.0)
    jk_out_ref[...] = jnp.maximum(jk_ref[...], xl)
    m = jnp.dot(xl, w_ref[...], preferred_element_type=jnp.float32)
    m_ref[...] = m
    mh_ref[...] = m * dinv


def _tc_mid(p, m_prev, dinv, b, jk, W):
    return pl.pallas_call(
        _mid_body,
        grid=(TC_GRID,),
        in_specs=[
            pl.BlockSpec((NC, ROW_BLK, D), lambda i: (0, i, 0)),
            pl.BlockSpec((ROW_BLK, D), lambda i: (i, 0)),
            pl.BlockSpec((ROW_BLK, 1), lambda i: (i, 0)),
            pl.BlockSpec((D,), lambda i: (0,)),
            pl.BlockSpec((ROW_BLK, D), lambda i: (i, 0)),
            pl.BlockSpec((D, D), lambda i: (0, 0)),
        ],
        out_specs=[
            pl.BlockSpec((ROW_BLK, D), lambda i: (i, 0)),
            pl.BlockSpec((ROW_BLK, D), lambda i: (i, 0)),
            pl.BlockSpec((ROW_BLK, D), lambda i: (i, 0)),
        ],
        out_shape=[
            jax.ShapeDtypeStruct((N_PAD, D), jnp.float32),
            jax.ShapeDtypeStruct((N_PAD, D), jnp.float32),
            jax.ShapeDtypeStruct((N_PAD, D), jnp.float32),
        ],
    )(p, m_prev, dinv, b, jk, W)


def _post_body(p_ref, m_prev_ref, dinv_ref, b_ref, jk_ref, wo_ref, bo_ref,
               out_ref):
    dinv = dinv_ref[...]
    agg = p_ref[0] + p_ref[1]
    xl = jnp.maximum(dinv * agg + dinv * dinv * m_prev_ref[...]
                     + b_ref[...][None, :], 0.0)
    jk = jnp.maximum(jk_ref[...], xl)
    out_ref[...] = (jnp.dot(jk, wo_ref[...], preferred_element_type=jnp.float32)
                    + bo_ref[...][None, :])


def _tc_post(p, m_prev, dinv, b, jk, Wo, bo):
    return pl.pallas_call(
        _post_body,
        grid=(TC_GRID,),
        in_specs=[
            pl.BlockSpec((NC, ROW_BLK, D), lambda i: (0, i, 0)),
            pl.BlockSpec((ROW_BLK, D), lambda i: (i, 0)),
            pl.BlockSpec((ROW_BLK, 1), lambda i: (i, 0)),
            pl.BlockSpec((D,), lambda i: (0,)),
            pl.BlockSpec((ROW_BLK, D), lambda i: (i, 0)),
            pl.BlockSpec((D, N_CLASS), lambda i: (0, 0)),
            pl.BlockSpec((N_CLASS,), lambda i: (0,)),
        ],
        out_specs=[pl.BlockSpec((ROW_BLK, N_CLASS), lambda i: (i, 0))],
        out_shape=[jax.ShapeDtypeStruct((N_PAD, N_CLASS), jnp.float32)],
    )(p, m_prev, dinv, b, jk, Wo, bo)


def kernel(x, edge_index, W0, b0, W1, b1, W2, b2, Wo, bo):
    src1 = edge_index[0]
    dst2 = edge_index[1].reshape(NC * NS, NCHUNK, CHUNK)
    x_pad = jnp.pad(x, ((0, N_PAD - N_NODES), (0, 0)))
    zeros_pad = jnp.zeros((N_PAD, D), jnp.float32)
    ones_rows = jnp.ones((CHUNK, DEG_W), jnp.float32)
    jk0 = jnp.zeros((N_PAD, D), jnp.float32)

    m0 = _tc_m0(x_pad, W0)[0]
    degp = _deg_kernel(dst2, zeros_pad, ones_rows).reshape(NC, N_PAD, DEG_W)
    mh0, dinv = _tc_scale(m0, degp)

    p1 = _agg_kernel(src1, dst2, mh0, zeros_pad).reshape(NC, N_PAD, D)
    m1, mh1, jk1 = _tc_mid(p1, m0, dinv, b0, jk0, W1)

    p2 = _agg_kernel(src1, dst2, mh1, zeros_pad).reshape(NC, N_PAD, D)
    m2, mh2, jk2 = _tc_mid(p2, m1, dinv, b1, jk1, W2)

    p3 = _agg_kernel(src1, dst2, mh2, zeros_pad).reshape(NC, N_PAD, D)
    out = _tc_post(p3, m2, dinv, b2, jk2, Wo, bo)[0]

    return out[:N_NODES]


# speedup vs baseline: 1.2662x; 1.0535x over previous
import jax
import jax.numpy as jnp
from jax import lax
from jax.experimental import pallas as pl
from jax.experimental.pallas import tpu as pltpu
from jax.experimental.pallas import tpu_sc as plsc

N_NODES = 10000
N_PAD = 10240
E = 320000
D = 128
N_CLASS = 40

NC = 2
NS = 16
EPT = E // (NC * NS)
CHUNK = 128
NFULL = EPT // CHUNK
TAIL = EPT - NFULL * CHUNK
NCHUNK = NFULL + 1
EPT_PAD = NCHUNK * CHUNK
STG1 = 40
STG2 = NFULL - STG1
SL = N_PAD // NS

ROW_BLK = 2560
TC_GRID = N_PAD // ROW_BLK


DEG_W = 128


def _deg_body(dst2_hbm, dtail_hbm, zeros_hbm, ones_hbm, deg_out,
              dstbuf, dtail, onesbuf, degacc, sem):
    c = lax.axis_index("c")
    s = lax.axis_index("s")
    g = c * NS + s

    pltpu.sync_copy(ones_hbm, onesbuf)
    pltpu.sync_copy(dst2_hbm.at[g], dstbuf)
    pltpu.sync_copy(dtail_hbm.at[g], dtail)
    pltpu.sync_copy(zeros_hbm.at[pl.ds(s * SL, SL)],
                    degacc.at[pl.ds(s * SL, SL)])
    plsc.subcore_barrier()

    def fire_body(j, _):
        pltpu.async_copy(onesbuf, degacc.at[dstbuf.at[j]], sem, add=True)
        return 0
    lax.fori_loop(0, NFULL, fire_body, 0)
    pltpu.async_copy(onesbuf.at[pl.ds(0, TAIL)], degacc.at[dtail.at[0]],
                     sem, add=True)

    def drain_body(j, _):
        pltpu.make_async_copy(onesbuf, degacc.at[dstbuf.at[j]], sem).wait()
        return 0
    lax.fori_loop(0, NFULL, drain_body, 0)
    pltpu.make_async_copy(onesbuf.at[pl.ds(0, TAIL)], degacc.at[dtail.at[0]],
                          sem).wait()

    plsc.subcore_barrier()
    pltpu.sync_copy(degacc.at[pl.ds(s * SL, SL)],
                    deg_out.at[pl.ds(c * N_PAD + s * SL, SL)])


def _deg_kernel(dst2, dtail, zeros_pad, ones_rows):
    mesh = plsc.VectorSubcoreMesh(core_axis_name="c", subcore_axis_name="s")
    return pl.kernel(
        _deg_body,
        out_type=jax.ShapeDtypeStruct((NC * N_PAD, DEG_W), jnp.float32),
        mesh=mesh,
        scratch_types=[
            pltpu.VMEM((NCHUNK, CHUNK), jnp.int32),
            pltpu.VMEM((1, TAIL), jnp.int32),
            pltpu.VMEM((CHUNK, DEG_W), jnp.float32),
            pltpu.VMEM_SHARED((N_PAD, DEG_W), jnp.float32),
            pltpu.SemaphoreType.DMA,
        ],
    )(dst2, dtail, zeros_pad, ones_rows)


def _agg_body(src1_hbm, dst2_hbm, dtail_hbm, mh_hbm, zeros_hbm, out_hbm,
              srcbuf, dstbuf, dtail, rows_a, rows_b, acc, sem_ga, sem_gb):
    c = lax.axis_index("c")
    s = lax.axis_index("s")
    g = c * NS + s

    pltpu.sync_copy(dst2_hbm.at[g], dstbuf)
    pltpu.sync_copy(dtail_hbm.at[g], dtail)
    pltpu.sync_copy(zeros_hbm.at[pl.ds(s * SL, SL)], acc.at[pl.ds(s * SL, SL)])
    plsc.subcore_barrier()

    def fire_g(o, rbuf, sem):
        pltpu.async_copy(mh_hbm.at[srcbuf.at[pl.ds(o, CHUNK)]], rbuf, sem)

    def wait_g(rbuf, sem):
        pltpu.make_async_copy(mh_hbm.at[srcbuf.at[pl.ds(0, CHUNK)]],
                              rbuf, sem).wait()

    def scatter(k, rbuf):
        pltpu.sync_copy(rbuf, acc.at[dstbuf.at[k]], add=True)

    def run_stage(cbase, npairs):
        fire_g(0, rows_a, sem_ga)

        def pair_body(i, _):
            fire_g((2 * i + 1) * CHUNK, rows_b, sem_gb)
            wait_g(rows_a, sem_ga)
            scatter(cbase + 2 * i, rows_a)

            @pl.when(i < npairs - 1)
            def _():
                fire_g((2 * i + 2) * CHUNK, rows_a, sem_ga)
            wait_g(rows_b, sem_gb)
            scatter(cbase + 2 * i + 1, rows_b)
            return 0
        lax.fori_loop(0, npairs, pair_body, 0)

    pltpu.sync_copy(src1_hbm.at[pl.ds(g * EPT_PAD, STG1 * CHUNK)], srcbuf)
    run_stage(0, STG1 // 2)
    pltpu.sync_copy(src1_hbm.at[pl.ds(g * EPT_PAD + STG1 * CHUNK,
                                      STG2 * CHUNK)],
                    srcbuf.at[pl.ds(0, STG2 * CHUNK)])
    run_stage(STG1, STG2 // 2)
    pltpu.sync_copy(src1_hbm.at[pl.ds(g * EPT_PAD + NFULL * CHUNK, TAIL)],
                    srcbuf.at[pl.ds(0, TAIL)])
    pltpu.async_copy(mh_hbm.at[srcbuf.at[pl.ds(0, TAIL)]],
                     rows_a.at[pl.ds(0, TAIL)], sem_ga)
    pltpu.make_async_copy(mh_hbm.at[srcbuf.at[pl.ds(0, TAIL)]],
                          rows_a.at[pl.ds(0, TAIL)], sem_ga).wait()
    pltpu.sync_copy(rows_a.at[pl.ds(0, TAIL)], acc.at[dtail.at[0]], add=True)

    plsc.subcore_barrier()
    pltpu.sync_copy(acc.at[pl.ds(s * SL, SL)],
                    out_hbm.at[pl.ds(c * N_PAD + s * SL, SL)])


def _agg_kernel(src1, dst2, dtail, mh, zeros_pad):
    mesh = plsc.VectorSubcoreMesh(core_axis_name="c", subcore_axis_name="s")
    return pl.kernel(
        _agg_body,
        out_type=jax.ShapeDtypeStruct((NC * N_PAD, D), jnp.float32),
        mesh=mesh,
        scratch_types=[
            pltpu.VMEM((STG1 * CHUNK,), jnp.int32),
            pltpu.VMEM((NCHUNK, CHUNK), jnp.int32),
            pltpu.VMEM((1, TAIL), jnp.int32),
            pltpu.VMEM((CHUNK, D), jnp.float32),
            pltpu.VMEM((CHUNK, D), jnp.float32),
            pltpu.VMEM_SHARED((N_PAD, D), jnp.float32),
            pltpu.SemaphoreType.DMA,
            pltpu.SemaphoreType.DMA,
        ],
    )(src1, dst2, dtail, mh, zeros_pad)


def _m0_body(x_ref, w_ref, m_ref):
    m_ref[...] = jnp.dot(x_ref[...], w_ref[...],
                         preferred_element_type=jnp.float32)


def _tc_m0(x_pad, W0):
    return pl.pallas_call(
        _m0_body,
        grid=(TC_GRID,),
        in_specs=[
            pl.BlockSpec((ROW_BLK, D), lambda i: (i, 0)),
            pl.BlockSpec((D, D), lambda i: (0, 0)),
        ],
        out_specs=[pl.BlockSpec((ROW_BLK, D), lambda i: (i, 0))],
        out_shape=[jax.ShapeDtypeStruct((N_PAD, D), jnp.float32)],
    )(x_pad, W0)


def _scale_body(m_ref, degp_ref, mh_ref, dinv_ref):
    deg = degp_ref[0, :, 0] + degp_ref[1, :, 0] + 1.0
    dinv = lax.rsqrt(deg)[:, None]
    mh_ref[...] = m_ref[...] * dinv
    dinv_ref[...] = dinv


def _tc_scale(m0, degp):
    return pl.pallas_call(
        _scale_body,
        grid=(TC_GRID,),
        in_specs=[
            pl.BlockSpec((ROW_BLK, D), lambda i: (i, 0)),
            pl.BlockSpec((NC, ROW_BLK, DEG_W), lambda i: (0, i, 0)),
        ],
        out_specs=[
            pl.BlockSpec((ROW_BLK, D), lambda i: (i, 0)),
            pl.BlockSpec((ROW_BLK, 1), lambda i: (i, 0)),
        ],
        out_shape=[
            jax.ShapeDtypeStruct((N_PAD, D), jnp.float32),
            jax.ShapeDtypeStruct((N_PAD, 1), jnp.float32),
        ],
    )(m0, degp)


def _mid_body(p_ref, m_prev_ref, dinv_ref, b_ref, jk_ref, w_ref,
              m_ref, mh_ref, jk_out_ref):
    dinv = dinv_ref[...]
    agg = p_ref[0] + p_ref[1]
    xl = jnp.maximum(dinv * agg + dinv * dinv * m_prev_ref[...]
                     + b_ref[...][None, :], 0.0)
    jk_out_ref[...] = jnp.maximum(jk_ref[...], xl)
    m = jnp.dot(xl, w_ref[...], preferred_element_type=jnp.float32)
    m_ref[...] = m
    mh_ref[...] = m * dinv


def _tc_mid(p, m_prev, dinv, b, jk, W):
    return pl.pallas_call(
        _mid_body,
        grid=(TC_GRID,),
        in_specs=[
            pl.BlockSpec((NC, ROW_BLK, D), lambda i: (0, i, 0)),
            pl.BlockSpec((ROW_BLK, D), lambda i: (i, 0)),
            pl.BlockSpec((ROW_BLK, 1), lambda i: (i, 0)),
            pl.BlockSpec((D,), lambda i: (0,)),
            pl.BlockSpec((ROW_BLK, D), lambda i: (i, 0)),
            pl.BlockSpec((D, D), lambda i: (0, 0)),
        ],
        out_specs=[
            pl.BlockSpec((ROW_BLK, D), lambda i: (i, 0)),
            pl.BlockSpec((ROW_BLK, D), lambda i: (i, 0)),
            pl.BlockSpec((ROW_BLK, D), lambda i: (i, 0)),
        ],
        out_shape=[
            jax.ShapeDtypeStruct((N_PAD, D), jnp.float32),
            jax.ShapeDtypeStruct((N_PAD, D), jnp.float32),
            jax.ShapeDtypeStruct((N_PAD, D), jnp.float32),
        ],
    )(p, m_prev, dinv, b, jk, W)


def _post_body(p_ref, m_prev_ref, dinv_ref, b_ref, jk_ref, wo_ref, bo_ref,
               out_ref):
    dinv = dinv_ref[...]
    agg = p_ref[0] + p_ref[1]
    xl = jnp.maximum(dinv * agg + dinv * dinv * m_prev_ref[...]
                     + b_ref[...][None, :], 0.0)
    jk = jnp.maximum(jk_ref[...], xl)
    out_ref[...] = (jnp.dot(jk, wo_ref[...], preferred_element_type=jnp.float32)
                    + bo_ref[...][None, :])


def _tc_post(p, m_prev, dinv, b, jk, Wo, bo):
    return pl.pallas_call(
        _post_body,
        grid=(TC_GRID,),
        in_specs=[
            pl.BlockSpec((NC, ROW_BLK, D), lambda i: (0, i, 0)),
            pl.BlockSpec((ROW_BLK, D), lambda i: (i, 0)),
            pl.BlockSpec((ROW_BLK, 1), lambda i: (i, 0)),
            pl.BlockSpec((D,), lambda i: (0,)),
            pl.BlockSpec((ROW_BLK, D), lambda i: (i, 0)),
            pl.BlockSpec((D, N_CLASS), lambda i: (0, 0)),
            pl.BlockSpec((N_CLASS,), lambda i: (0,)),
        ],
        out_specs=[pl.BlockSpec((ROW_BLK, N_CLASS), lambda i: (i, 0))],
        out_shape=[jax.ShapeDtypeStruct((N_PAD, N_CLASS), jnp.float32)],
    )(p, m_prev, dinv, b, jk, Wo, bo)


def kernel(x, edge_index, W0, b0, W1, b1, W2, b2, Wo, bo):
    src_t = edge_index[0].reshape(NC * NS, EPT)
    dst_t = edge_index[1].reshape(NC * NS, EPT)
    src1 = jnp.pad(src_t, ((0, 0), (0, EPT_PAD - EPT))).reshape(-1)
    dst2 = jnp.pad(dst_t, ((0, 0), (0, EPT_PAD - EPT))).reshape(
        NC * NS, NCHUNK, CHUNK)
    dtail = dst_t[:, NFULL * CHUNK:].reshape(NC * NS, 1, TAIL)
    x_pad = jnp.pad(x, ((0, N_PAD - N_NODES), (0, 0)))
    zeros_pad = jnp.zeros((N_PAD, D), jnp.float32)
    ones_rows = jnp.ones((CHUNK, DEG_W), jnp.float32)
    jk0 = jnp.zeros((N_PAD, D), jnp.float32)

    m0 = _tc_m0(x_pad, W0)[0]
    degp = _deg_kernel(dst2, dtail, zeros_pad,
                       ones_rows).reshape(NC, N_PAD, DEG_W)
    mh0, dinv = _tc_scale(m0, degp)

    p1 = _agg_kernel(src1, dst2, dtail, mh0, zeros_pad).reshape(NC, N_PAD, D)
    m1, mh1, jk1 = _tc_mid(p1, m0, dinv, b0, jk0, W1)

    p2 = _agg_kernel(src1, dst2, dtail, mh1, zeros_pad).reshape(NC, N_PAD, D)
    m2, mh2, jk2 = _tc_mid(p2, m1, dinv, b1, jk1, W2)

    p3 = _agg_kernel(src1, dst2, dtail, mh2, zeros_pad).reshape(NC, N_PAD, D)
    out = _tc_post(p3, m2, dinv, b2, jk2, Wo, bo)[0]

    return out[:N_NODES]
